# baseline (speedup 1.0000x reference)
import functools

import jax
import jax.numpy as jnp
from jax import lax
from jax.experimental import pallas as pl
from jax.experimental.pallas import tpu as pltpu
from jax.experimental.pallas import tpu_sc as plsc

N = 10000
D = 128
ALPHA = 0.5

NC = 2
NS = 16
NW = NC * NS
CH = 80
CHUNKS = 126
NPAD = 10240
RPD = NPAD // NS

_MESH = dict(core_axis_name="c", subcore_axis_name="s")


@functools.partial(
    pl.kernel,
    out_type=jax.ShapeDtypeStruct((NC, NS, RPD), jnp.float32),
    mesh=plsc.VectorSubcoreMesh(**_MESH),
    scratch_types=[
        pltpu.VMEM((CHUNKS, CH), jnp.int32),
        pltpu.VMEM((CH,), jnp.float32),
        pltpu.VMEM((RPD,), jnp.float32),
        pltpu.VMEM_SHARED((NPAD,), jnp.float32),
    ],
)
def _deg_kernel(dst_hbm, out_hbm, idx_v, ones_v, zb_v, deg_sh):
  c = lax.axis_index("c")
  s = lax.axis_index("s")
  wid = c * NS + s
  pltpu.sync_copy(dst_hbm.at[wid], idx_v)

  def zb_body(i, _):
    zb_v[pl.ds(i * 16, 16)] = jnp.zeros((16,), jnp.float32)
    return 0

  lax.fori_loop(0, RPD // 16, zb_body, 0)

  def ones_body(i, _):
    ones_v[pl.ds(i * 16, 16)] = jnp.ones((16,), jnp.float32)
    return 0

  lax.fori_loop(0, CH // 16, ones_body, 0)

  pltpu.sync_copy(zb_v, deg_sh.at[pl.ds(s * RPD, RPD)])
  plsc.subcore_barrier()

  def edge_body(j, _):
    pltpu.sync_copy(ones_v, deg_sh.at[idx_v.at[j]], add=True)
    return 0

  lax.fori_loop(0, CHUNKS, edge_body, 0)
  plsc.subcore_barrier()
  pltpu.sync_copy(deg_sh.at[pl.ds(s * RPD, RPD)], out_hbm.at[c, s])


@functools.partial(
    pl.kernel,
    out_type=jax.ShapeDtypeStruct((NC, NPAD, D), jnp.float32),
    mesh=plsc.VectorSubcoreMesh(**_MESH),
    scratch_types=[
        pltpu.VMEM((2, CHUNKS, CH), jnp.int32),
        pltpu.VMEM((CH, D), jnp.float32),
        pltpu.VMEM_SHARED((NPAD, D), jnp.float32),
    ],
)
def _edge_kernel(g_hbm, ei_hbm, out_hbm, ei_v, rows_v, t_sh):
  c = lax.axis_index("c")
  s = lax.axis_index("s")
  wid = c * NS + s
  pltpu.sync_copy(ei_hbm.at[wid], ei_v)

  def zero_body(i, _):
    r = i // (D // 16)
    q = i - r * (D // 16)
    rows_v[r, pl.ds(q * 16, 16)] = jnp.zeros((16,), jnp.float32)
    return 0

  lax.fori_loop(0, CH * (D // 16), zero_body, 0)
  for m in range(RPD // CH):
    pltpu.sync_copy(rows_v, t_sh.at[pl.ds(s * RPD + m * CH, CH)])
  plsc.subcore_barrier()

  def edge_body(j, _):
    pltpu.sync_copy(g_hbm.at[ei_v.at[0, j]], rows_v)
    pltpu.sync_copy(rows_v, t_sh.at[ei_v.at[1, j]], add=True)
    return 0

  lax.fori_loop(0, CHUNKS, edge_body, 0)
  plsc.subcore_barrier()
  pltpu.sync_copy(t_sh.at[pl.ds(s * RPD, RPD)], out_hbm.at[c, pl.ds(s * RPD, RPD)])


BM = 2000


def _g_body(deg_ref, x_ref, w_ref, g_ref):
  dinv = lax.rsqrt(deg_ref[0] + deg_ref[1] + 1.0)
  h = jnp.dot(x_ref[...], w_ref[...], preferred_element_type=jnp.float32)
  g_ref[...] = dinv * h


def _fin_body(deg_ref, t_ref, g_ref, x_ref, b_ref, o_ref):
  dinv = lax.rsqrt(deg_ref[0] + deg_ref[1] + 1.0)
  conv = dinv * (t_ref[0] + t_ref[1] + g_ref[...]) + b_ref[...]
  o_ref[...] = jnp.maximum(ALPHA * conv + (1.0 - ALPHA) * x_ref[...], 0.0)


def kernel(x, edge_index, W, b):
  e = edge_index.shape[1]
  src = edge_index[0].astype(jnp.int32)
  dst = edge_index[1].astype(jnp.int32)
  pad = NW * CHUNKS * CH - e
  src_p = jnp.concatenate([src, jnp.arange(pad, dtype=jnp.int32) % N])
  dst_p = jnp.concatenate(
      [dst, N + jnp.arange(pad, dtype=jnp.int32) % (NPAD - N)])
  src_p = src_p.reshape(NW, CHUNKS, CH)
  dst_p = dst_p.reshape(NW, CHUNKS, CH)

  ei_p = jnp.stack([src_p, dst_p], axis=1)

  deg2 = _deg_kernel(dst_p).reshape(NC, NPAD, 1)

  grid = N // BM
  g = pl.pallas_call(
      _g_body,
      grid=(grid,),
      in_specs=[
          pl.BlockSpec((NC, BM, 1), lambda i: (0, i, 0)),
          pl.BlockSpec((BM, D), lambda i: (i, 0)),
          pl.BlockSpec((D, D), lambda i: (0, 0)),
      ],
      out_specs=pl.BlockSpec((BM, D), lambda i: (i, 0)),
      out_shape=jax.ShapeDtypeStruct((N, D), jnp.float32),
  )(deg2, x, W)

  t_part = _edge_kernel(g, ei_p)

  out = pl.pallas_call(
      _fin_body,
      grid=(grid,),
      in_specs=[
          pl.BlockSpec((NC, BM, 1), lambda i: (0, i, 0)),
          pl.BlockSpec((NC, BM, D), lambda i: (0, i, 0)),
          pl.BlockSpec((BM, D), lambda i: (i, 0)),
          pl.BlockSpec((BM, D), lambda i: (i, 0)),
          pl.BlockSpec((1, D), lambda i: (0, 0)),
      ],
      out_specs=pl.BlockSpec((BM, D), lambda i: (i, 0)),
      out_shape=jax.ShapeDtypeStruct((N, D), jnp.float32),
  )(deg2, t_part, g, x, b.reshape(1, D))
  return out


# speedup vs baseline: 1.1387x; 1.1387x over previous
import functools

import jax
import jax.numpy as jnp
from jax import lax
from jax.experimental import pallas as pl
from jax.experimental.pallas import tpu as pltpu
from jax.experimental.pallas import tpu_sc as plsc

N = 10000
D = 128
ALPHA = 0.5

NC = 2
NS = 16
NW = NC * NS
CH = 128
CHUNKS = 80
NPAD = 10240
RPD = NPAD // NS

_MESH = dict(core_axis_name="c", subcore_axis_name="s")


@functools.partial(
    pl.kernel,
    out_type=jax.ShapeDtypeStruct((NC, NS, RPD), jnp.float32),
    mesh=plsc.VectorSubcoreMesh(**_MESH),
    scratch_types=[
        pltpu.VMEM((CHUNKS, CH), jnp.int32),
        pltpu.VMEM((CH,), jnp.float32),
        pltpu.VMEM((RPD,), jnp.float32),
        pltpu.VMEM_SHARED((NPAD,), jnp.float32),
    ],
)
def _deg_kernel(dst_hbm, out_hbm, idx_v, ones_v, zb_v, deg_sh):
  c = lax.axis_index("c")
  s = lax.axis_index("s")
  wid = c * NS + s
  pltpu.sync_copy(dst_hbm.at[wid], idx_v)

  def zb_body(i, _):
    zb_v[pl.ds(i * 16, 16)] = jnp.zeros((16,), jnp.float32)
    return 0

  lax.fori_loop(0, RPD // 16, zb_body, 0)

  def ones_body(i, _):
    ones_v[pl.ds(i * 16, 16)] = jnp.ones((16,), jnp.float32)
    return 0

  lax.fori_loop(0, CH // 16, ones_body, 0)

  pltpu.sync_copy(zb_v, deg_sh.at[pl.ds(s * RPD, RPD)])
  plsc.subcore_barrier()

  def edge_body(j, _):
    pltpu.sync_copy(ones_v, deg_sh.at[idx_v.at[j]], add=True)
    return 0

  lax.fori_loop(0, CHUNKS, edge_body, 0)
  plsc.subcore_barrier()
  pltpu.sync_copy(deg_sh.at[pl.ds(s * RPD, RPD)], out_hbm.at[c, s])


@functools.partial(
    pl.kernel,
    out_type=jax.ShapeDtypeStruct((NC, NPAD, D), jnp.float32),
    mesh=plsc.VectorSubcoreMesh(**_MESH),
    scratch_types=[
        pltpu.VMEM((2, CHUNKS, CH), jnp.int32),
        pltpu.VMEM((CH, D), jnp.float32),
        pltpu.VMEM_SHARED((NPAD, D), jnp.float32),
    ],
)
def _edge_kernel(g_hbm, ei_hbm, out_hbm, ei_v, rows_v, t_sh):
  c = lax.axis_index("c")
  s = lax.axis_index("s")
  wid = c * NS + s
  pltpu.sync_copy(ei_hbm.at[wid], ei_v)

  def zero_body(i, _):
    r = i // (D // 16)
    q = i - r * (D // 16)
    rows_v[r, pl.ds(q * 16, 16)] = jnp.zeros((16,), jnp.float32)
    return 0

  lax.fori_loop(0, CH * (D // 16), zero_body, 0)
  for m in range(RPD // CH):
    pltpu.sync_copy(rows_v, t_sh.at[pl.ds(s * RPD + m * CH, CH)])
  plsc.subcore_barrier()

  def edge_body(j, _):
    pltpu.sync_copy(g_hbm.at[ei_v.at[0, j]], rows_v)
    pltpu.sync_copy(rows_v, t_sh.at[ei_v.at[1, j]], add=True)
    return 0

  lax.fori_loop(0, CHUNKS, edge_body, 0)
  plsc.subcore_barrier()
  pltpu.sync_copy(t_sh.at[pl.ds(s * RPD, RPD)], out_hbm.at[c, pl.ds(s * RPD, RPD)])


BM = 2000


def _g_body(deg_ref, x_ref, w_ref, g_ref):
  dinv = lax.rsqrt(deg_ref[0] + deg_ref[1] + 1.0)
  h = jnp.dot(x_ref[...], w_ref[...], preferred_element_type=jnp.float32)
  g_ref[...] = dinv * h


def _fin_body(deg_ref, t_ref, g_ref, x_ref, b_ref, o_ref):
  dinv = lax.rsqrt(deg_ref[0] + deg_ref[1] + 1.0)
  conv = dinv * (t_ref[0] + t_ref[1] + g_ref[...]) + b_ref[...]
  o_ref[...] = jnp.maximum(ALPHA * conv + (1.0 - ALPHA) * x_ref[...], 0.0)


def kernel(x, edge_index, W, b):
  e = edge_index.shape[1]
  src = edge_index[0].astype(jnp.int32)
  dst = edge_index[1].astype(jnp.int32)
  pad = NW * CHUNKS * CH - e
  src_p = jnp.concatenate([src, jnp.arange(pad, dtype=jnp.int32) % N])
  dst_p = jnp.concatenate(
      [dst, N + jnp.arange(pad, dtype=jnp.int32) % (NPAD - N)])
  src_p = src_p.reshape(NW, CHUNKS, CH)
  dst_p = dst_p.reshape(NW, CHUNKS, CH)

  ei_p = jnp.stack([src_p, dst_p], axis=1)

  deg2 = _deg_kernel(dst_p).reshape(NC, NPAD, 1)

  grid = N // BM
  g = pl.pallas_call(
      _g_body,
      grid=(grid,),
      in_specs=[
          pl.BlockSpec((NC, BM, 1), lambda i: (0, i, 0)),
          pl.BlockSpec((BM, D), lambda i: (i, 0)),
          pl.BlockSpec((D, D), lambda i: (0, 0)),
      ],
      out_specs=pl.BlockSpec((BM, D), lambda i: (i, 0)),
      out_shape=jax.ShapeDtypeStruct((N, D), jnp.float32),
  )(deg2, x, W)

  t_part = _edge_kernel(g, ei_p)

  out = pl.pallas_call(
      _fin_body,
      grid=(grid,),
      in_specs=[
          pl.BlockSpec((NC, BM, 1), lambda i: (0, i, 0)),
          pl.BlockSpec((NC, BM, D), lambda i: (0, i, 0)),
          pl.BlockSpec((BM, D), lambda i: (i, 0)),
          pl.BlockSpec((BM, D), lambda i: (i, 0)),
          pl.BlockSpec((1, D), lambda i: (0, 0)),
      ],
      out_specs=pl.BlockSpec((BM, D), lambda i: (i, 0)),
      out_shape=jax.ShapeDtypeStruct((N, D), jnp.float32),
  )(deg2, t_part, g, x, b.reshape(1, D))
  return out
